# baseline (speedup 1.0000x reference)
import jax
import jax.numpy as jnp
from jax import lax
from jax.experimental import pallas as pl
from jax.experimental.pallas import tpu as pltpu
from jax.experimental.pallas import tpu_sc as plsc

NUM_CORES = 2
NUM_SUBCORES = 16
NUM_WORKERS = NUM_CORES * NUM_SUBCORES

B = 16384 * 50
D = 32
SUP = 128
N_SUP = 250000
B_PER_W = B // NUM_WORKERS
CHUNK = 256
N_CHUNKS = B_PER_W // CHUNK
G = CHUNK // 16
K = 4
SUBC = CHUNK // K


def _body(table_hbm, ids_hbm, out_hbm,
          ids_a, ids_b, sup_a, sup_b, rows_a, rows_b, out_v,
          sem_a, sem_b):
    wid = lax.axis_index("s") * NUM_CORES + lax.axis_index("c")
    base = wid * B_PER_W
    iota = lax.iota(jnp.int32, 16)

    def stage_and_fire(j, ids_v, sup_v, rows_v, sem):
        pltpu.sync_copy(ids_hbm.at[pl.ds(base + j * CHUNK, CHUNK)], ids_v)

        @plsc.parallel_loop(0, G, unroll=2)
        def _(g):
            ids16 = ids_v[pl.ds(g * 16, 16)]
            sup_v[pl.ds(g * 16, 16)] = lax.shift_right_logical(ids16, 2)

        for k in range(K):
            pltpu.async_copy(
                table_hbm.at[sup_v.at[pl.ds(k * SUBC, SUBC)]],
                rows_v.at[pl.ds(k * SUBC, SUBC)], sem)

    def drain(sup_v, rows_v, sem):
        for k in range(K):
            pltpu.make_async_copy(
                table_hbm.at[sup_v.at[pl.ds(k * SUBC, SUBC)]],
                rows_v.at[pl.ds(k * SUBC, SUBC)], sem).wait()

    def extract_and_store(j, ids_v, rows_v):
        @plsc.parallel_loop(0, G, unroll=2)
        def _(g):
            ids16 = ids_v[pl.ds(g * 16, 16)]
            col0 = lax.shift_left(lax.bitwise_and(ids16, 3), 5)
            row16 = g * 16 + iota
            vs = [plsc.load_gather(rows_v, [row16, col0 + d]) for d in range(D)]
            for d in range(D):
                plsc.store_scatter(out_v, [row16, jnp.full((16,), d, jnp.int32)], vs[d])

        pltpu.sync_copy(out_v, out_hbm.at[pl.ds(base + j * CHUNK, CHUNK)])

    stage_and_fire(0, ids_a, sup_a, rows_a, sem_a)

    @pl.loop(0, N_CHUNKS // 2)
    def _(i):
        j0 = 2 * i
        stage_and_fire(j0 + 1, ids_b, sup_b, rows_b, sem_b)
        drain(sup_a, rows_a, sem_a)
        extract_and_store(j0, ids_a, rows_a)

        @pl.when(j0 + 2 < N_CHUNKS)
        def _():
            stage_and_fire(j0 + 2, ids_a, sup_a, rows_a, sem_a)

        drain(sup_b, rows_b, sem_b)
        extract_and_store(j0 + 1, ids_b, rows_b)


def kernel(token_ids, embedding_matrix):
    flat_ids = token_ids.reshape(B).astype(jnp.int32)
    table_sup = embedding_matrix.reshape(N_SUP, SUP)

    mesh = plsc.VectorSubcoreMesh(core_axis_name="c", subcore_axis_name="s")
    lookup = pl.kernel(
        _body,
        mesh=mesh,
        out_type=jax.ShapeDtypeStruct((B, D), jnp.float32),
        scratch_types=[
            pltpu.VMEM((CHUNK,), jnp.int32),
            pltpu.VMEM((CHUNK,), jnp.int32),
            pltpu.VMEM((CHUNK,), jnp.int32),
            pltpu.VMEM((CHUNK,), jnp.int32),
            pltpu.VMEM((CHUNK, SUP), jnp.float32),
            pltpu.VMEM((CHUNK, SUP), jnp.float32),
            pltpu.VMEM((CHUNK, D), jnp.float32),
            pltpu.SemaphoreType.DMA,
            pltpu.SemaphoreType.DMA,
        ],
        compiler_params=pltpu.CompilerParams(needs_layout_passes=False),
    )
    out = lookup(table_sup, flat_ids)
    return out.reshape(token_ids.shape + (D,))


# speedup vs baseline: 1.0091x; 1.0091x over previous
import jax
import jax.numpy as jnp
from jax import lax
from jax.experimental import pallas as pl
from jax.experimental.pallas import tpu as pltpu
from jax.experimental.pallas import tpu_sc as plsc

NUM_CORES = 2
NUM_SUBCORES = 16
NUM_WORKERS = NUM_CORES * NUM_SUBCORES

B = 16384 * 50
D = 32
SUP = 128
N_SUP = 250000
B_PER_W = B // NUM_WORKERS
CHUNK = 256
N_CHUNKS = B_PER_W // CHUNK
G = CHUNK // 16
K = 4
SUBC = CHUNK // K


def _body(table_hbm, ids_hbm, out_hbm,
          ids_a, ids_b, sup_a, sup_b, rows_a, rows_b, out_v,
          sem_a, sem_b):
    wid = lax.axis_index("s") * NUM_CORES + lax.axis_index("c")
    base = wid * B_PER_W
    iota = lax.iota(jnp.int32, 16)

    def stage_and_fire(j, ids_v, sup_v, rows_v, sem):
        pltpu.sync_copy(ids_hbm.at[pl.ds(base + j * CHUNK, CHUNK)], ids_v)

        @pl.loop(0, G)
        def _(g):
            ids16 = ids_v[pl.ds(g * 16, 16)]
            sup_v[pl.ds(g * 16, 16)] = lax.shift_right_logical(ids16, 2)

        for k in range(K):
            pltpu.async_copy(
                table_hbm.at[sup_v.at[pl.ds(k * SUBC, SUBC)]],
                rows_v.at[pl.ds(k * SUBC, SUBC)], sem)

    def drain_one(sup_v, rows_v, sem, k):
        pltpu.make_async_copy(
            table_hbm.at[sup_v.at[pl.ds(k * SUBC, SUBC)]],
            rows_v.at[pl.ds(k * SUBC, SUBC)], sem).wait()

    GK = G // K

    def extract_and_store(j, ids_v, sup_v, rows_v, sem):
        for k in range(K):
            drain_one(sup_v, rows_v, sem, k)

            @pl.loop(k * GK, (k + 1) * GK)
            def _(g):
                ids16 = ids_v[pl.ds(g * 16, 16)]
                col0 = lax.shift_left(lax.bitwise_and(ids16, 3), 5)
                row16 = g * 16 + iota
                vs = [plsc.load_gather(rows_v, [row16, col0 + d]) for d in range(D)]
                for d in range(D):
                    plsc.store_scatter(out_v, [row16, jnp.full((16,), d, jnp.int32)], vs[d])

        pltpu.sync_copy(out_v, out_hbm.at[pl.ds(base + j * CHUNK, CHUNK)])

    stage_and_fire(0, ids_a, sup_a, rows_a, sem_a)

    @pl.loop(0, N_CHUNKS // 2)
    def _(i):
        j0 = 2 * i
        stage_and_fire(j0 + 1, ids_b, sup_b, rows_b, sem_b)
        extract_and_store(j0, ids_a, sup_a, rows_a, sem_a)

        @pl.when(j0 + 2 < N_CHUNKS)
        def _():
            stage_and_fire(j0 + 2, ids_a, sup_a, rows_a, sem_a)

        extract_and_store(j0 + 1, ids_b, sup_b, rows_b, sem_b)


def kernel(token_ids, embedding_matrix):
    flat_ids = token_ids.reshape(B).astype(jnp.int32)
    table_sup = embedding_matrix.reshape(N_SUP, SUP)

    mesh = plsc.VectorSubcoreMesh(core_axis_name="c", subcore_axis_name="s")
    lookup = pl.kernel(
        _body,
        mesh=mesh,
        out_type=jax.ShapeDtypeStruct((B, D), jnp.float32),
        scratch_types=[
            pltpu.VMEM((CHUNK,), jnp.int32),
            pltpu.VMEM((CHUNK,), jnp.int32),
            pltpu.VMEM((CHUNK,), jnp.int32),
            pltpu.VMEM((CHUNK,), jnp.int32),
            pltpu.VMEM((CHUNK, SUP), jnp.float32),
            pltpu.VMEM((CHUNK, SUP), jnp.float32),
            pltpu.VMEM((CHUNK, D), jnp.float32),
            pltpu.SemaphoreType.DMA,
            pltpu.SemaphoreType.DMA,
        ],
        compiler_params=pltpu.CompilerParams(needs_layout_passes=False),
    )
    out = lookup(table_sup, flat_ids)
    return out.reshape(token_ids.shape + (D,))
